# baseline (speedup 1.0000x reference)
import functools

import jax
import jax.numpy as jnp
from jax import lax
from jax.experimental import pallas as pl
from jax.experimental.pallas import tpu as pltpu
from jax.experimental.pallas import tpu_sc as plsc

DIM = 768
INTER = 768
E = 8
SINTER = 1536
NTOK = 2048

BM = 128
MAXB = 40
S = MAXB * BM

NW = 32
CHUNK = NTOK // NW
HALF = CHUNK // 2


def _mm_nt(a, b):
    return jax.lax.dot_general(
        a, b, (((1,), (1,)), ((), ())), preferred_element_type=jnp.float32)


def _gate_shared_kernel(x_ref, gw_ref, SW1_ref, SB1_ref, SW2_ref, SB2_ref,
                        SW3_ref, SB3_ref,
                        ysh_ref, pos0_ref, pos1_ref, w0_ref, w1_ref, be_ref):
    x = x_ref[...]
    scores = _mm_nt(x, gw_ref[...])
    m = jnp.max(scores, axis=-1, keepdims=True)
    p = jnp.exp(scores - m)
    s = p / jnp.sum(p, axis=-1, keepdims=True)
    lane = jax.lax.broadcasted_iota(jnp.int32, (NTOK, E), 1)
    m1 = jnp.max(s, axis=-1, keepdims=True)
    idx1 = jnp.min(jnp.where(s == m1, lane, E), axis=-1, keepdims=True)
    oh1 = lane == idx1
    s_m = jnp.where(oh1, -jnp.inf, s)
    m2 = jnp.max(s_m, axis=-1, keepdims=True)
    idx2 = jnp.min(jnp.where(s_m == m2, lane, E), axis=-1, keepdims=True)
    oh2 = lane == idx2
    sel = jnp.logical_or(oh1, oh2)

    msk = sel.astype(jnp.float32)
    acc = msk
    shift = 1
    while shift < NTOK:
        acc = acc + jnp.concatenate(
            [jnp.zeros((shift, E), jnp.float32), acc[:NTOK - shift]], axis=0)
        shift *= 2
    rank = (acc - msk).astype(jnp.int32)
    counts = acc[NTOK - 1:NTOK, :].astype(jnp.int32)
    pc = ((counts + (BM - 1)) // BM) * BM
    off = pc
    off = off + jnp.concatenate(
        [jnp.zeros((1, 1), jnp.int32), off[:, :E - 1]], axis=1)
    off = off + jnp.concatenate(
        [jnp.zeros((1, 2), jnp.int32), off[:, :E - 2]], axis=1)
    off = off + jnp.concatenate(
        [jnp.zeros((1, 4), jnp.int32), off[:, :E - 4]], axis=1)
    off = off - pc
    pos = off + rank
    pos0_ref[...] = jnp.sum(jnp.where(oh1, pos, 0), axis=-1, keepdims=True)
    pos1_ref[...] = jnp.sum(jnp.where(oh2, pos, 0), axis=-1, keepdims=True)
    w0 = jnp.sum(jnp.where(oh1, s, 0.0), axis=-1, keepdims=True)
    w1 = jnp.sum(jnp.where(oh2, s, 0.0), axis=-1, keepdims=True)
    w0_ref[...] = jnp.broadcast_to(w0, (NTOK, 16))
    w1_ref[...] = jnp.broadcast_to(w1, (NTOK, 16))
    bstart = jax.lax.broadcasted_iota(jnp.int32, (MAXB, E), 0) * BM
    ends = off + pc
    be = jnp.sum((bstart >= ends).astype(jnp.int32), axis=-1, keepdims=True)
    be_ref[...] = jnp.minimum(be, E - 1)
    h = jax.nn.silu(_mm_nt(x, SW1_ref[...]) + SB1_ref[...]) * (
        _mm_nt(x, SW3_ref[...]) + SB3_ref[...])
    ysh_ref[...] = _mm_nt(h, SW2_ref[...]) + SB2_ref[...]


def _gate_shared(x, gate_w, SW1, SB1, SW2, SB2, SW3, SB3):
    whole = lambda i: tuple(0 for _ in range(2))
    outs = pl.pallas_call(
        _gate_shared_kernel,
        in_specs=[pl.BlockSpec(a.shape, lambda i=0: (0, 0)) for a in
                  (x, gate_w, SW1, SB1, SW2, SB2, SW3, SB3)],
        out_specs=[
            pl.BlockSpec((NTOK, DIM), lambda: (0, 0)),
            pl.BlockSpec((NTOK, 1), lambda: (0, 0)),
            pl.BlockSpec((NTOK, 1), lambda: (0, 0)),
            pl.BlockSpec((NTOK, 16), lambda: (0, 0)),
            pl.BlockSpec((NTOK, 16), lambda: (0, 0)),
            pl.BlockSpec((MAXB, 1), lambda: (0, 0)),
        ],
        out_shape=[
            jax.ShapeDtypeStruct((NTOK, DIM), jnp.float32),
            jax.ShapeDtypeStruct((NTOK, 1), jnp.int32),
            jax.ShapeDtypeStruct((NTOK, 1), jnp.int32),
            jax.ShapeDtypeStruct((NTOK, 16), jnp.float32),
            jax.ShapeDtypeStruct((NTOK, 16), jnp.float32),
            jax.ShapeDtypeStruct((MAXB, 1), jnp.int32),
        ],
    )(x, gate_w, SW1, SB1, SW2, SB2, SW3, SB3)
    return outs


def _dispatch_kernel(x_hbm, posb_hbm, xs_hbm,
                     xb, idx0, idx1, sem0, sem1):
    wid = lax.axis_index("s") * 2 + lax.axis_index("c")
    base = wid * CHUNK
    pltpu.sync_copy(x_hbm.at[pl.ds(base, CHUNK)], xb)
    pltpu.sync_copy(posb_hbm.at[wid], idx0)
    pltpu.sync_copy(posb_hbm.at[NW + wid], idx1)
    c0 = pltpu.make_async_copy(xb, xs_hbm.at[idx0], sem0)
    c1 = pltpu.make_async_copy(xb, xs_hbm.at[idx1], sem1)
    c0.start(); c1.start()
    c0.wait(); c1.wait()


def _dispatch(x, posb):
    mesh = plsc.VectorSubcoreMesh(core_axis_name="c", subcore_axis_name="s")
    run = pl.kernel(
        _dispatch_kernel, mesh=mesh,
        out_type=jax.ShapeDtypeStruct((S, DIM), jnp.float32),
        scratch_types=[
            pltpu.VMEM((CHUNK, DIM), jnp.float32),
            pltpu.VMEM((CHUNK,), jnp.int32),
            pltpu.VMEM((CHUNK,), jnp.int32),
            pltpu.SemaphoreType.DMA,
            pltpu.SemaphoreType.DMA,
        ],
    )
    return run(x, posb)


def _grouped_kernel(be_ref, xs_ref, W1_ref, B1_ref, W2_ref, B2_ref,
                    W3_ref, B3_ref, out_ref):
    xs = xs_ref[...]
    h = jax.nn.silu(_mm_nt(xs, W1_ref[0]) + B1_ref[0]) * (
        _mm_nt(xs, W3_ref[0]) + B3_ref[0])
    out_ref[...] = _mm_nt(h, W2_ref[0]) + B2_ref[0]


def _grouped(be, xs, W1, B1, W2, B2, W3, B3):
    ew = lambda b, be_ref: (be_ref[b], 0, 0)
    grid_spec = pltpu.PrefetchScalarGridSpec(
        num_scalar_prefetch=1,
        grid=(MAXB,),
        in_specs=[
            pl.BlockSpec((BM, DIM), lambda b, be_ref: (b, 0)),
            pl.BlockSpec((1, INTER, DIM), ew),
            pl.BlockSpec((1, 1, INTER), ew),
            pl.BlockSpec((1, DIM, INTER), ew),
            pl.BlockSpec((1, 1, DIM), ew),
            pl.BlockSpec((1, INTER, DIM), ew),
            pl.BlockSpec((1, 1, INTER), ew),
        ],
        out_specs=pl.BlockSpec((BM, DIM), lambda b, be_ref: (b, 0)),
    )
    return pl.pallas_call(
        _grouped_kernel,
        grid_spec=grid_spec,
        out_shape=jax.ShapeDtypeStruct((S, DIM), jnp.float32),
        compiler_params=pltpu.CompilerParams(
            dimension_semantics=("arbitrary",)),
    )(be, xs, W1, B1.reshape(E, 1, INTER), W2, B2.reshape(E, 1, DIM),
      W3, B3.reshape(E, 1, INTER))


def _combine_kernel(ysh_hbm, op_hbm, posd_hbm, wd_hbm, y_hbm,
                    acc, r0, r1, idx0, idx1, wv0, wv1, sem0, sem1):
    wid = lax.axis_index("s") * 2 + lax.axis_index("c")

    def sub(j, _):
        base = wid * CHUNK + j * HALF
        pltpu.sync_copy(posd_hbm.at[wid * 2 + j], idx0)
        pltpu.sync_copy(posd_hbm.at[NW * 2 + wid * 2 + j], idx1)
        pltpu.sync_copy(wd_hbm.at[wid * 2 + j], wv0)
        pltpu.sync_copy(wd_hbm.at[NW * 2 + wid * 2 + j], wv1)
        pltpu.sync_copy(ysh_hbm.at[pl.ds(base, HALF)], acc)
        c0 = pltpu.make_async_copy(op_hbm.at[idx0], r0, sem0)
        c1 = pltpu.make_async_copy(op_hbm.at[idx1], r1, sem1)
        c0.start(); c1.start(); c0.wait(); c1.wait()

        def row(r, _):
            g0 = wv0[r, pl.ds(0, 16)]
            g1 = wv1[r, pl.ds(0, 16)]
            for c in range(DIM // 16):
                sl = pl.ds(c * 16, 16)
                acc[r, sl] = acc[r, sl] + g0 * r0[r, sl] + g1 * r1[r, sl]
            return 0
        lax.fori_loop(0, HALF, row, 0)
        pltpu.sync_copy(acc, y_hbm.at[pl.ds(base, HALF)])
        return 0

    lax.fori_loop(0, 2, sub, 0)


def _combine(ysh, op, posd, wd):
    mesh = plsc.VectorSubcoreMesh(core_axis_name="c", subcore_axis_name="s")
    run = pl.kernel(
        _combine_kernel, mesh=mesh,
        out_type=jax.ShapeDtypeStruct((NTOK, DIM), jnp.float32),
        scratch_types=[
            pltpu.VMEM((HALF, DIM), jnp.float32),
            pltpu.VMEM((HALF, DIM), jnp.float32),
            pltpu.VMEM((HALF, DIM), jnp.float32),
            pltpu.VMEM((HALF,), jnp.int32),
            pltpu.VMEM((HALF,), jnp.int32),
            pltpu.VMEM((HALF, 16), jnp.float32),
            pltpu.VMEM((HALF, 16), jnp.float32),
            pltpu.SemaphoreType.DMA,
            pltpu.SemaphoreType.DMA,
        ],
    )
    return run(ysh, op, posd, wd)


def kernel(x, gate_w, W1, B1, W2, B2, W3, B3, SW1, SB1, SW2, SB2, SW3, SB3):
    ysh, pos0, pos1, w0, w1, be = _gate_shared(
        x, gate_w, SW1, SB1.reshape(1, SINTER), SW2, SB2.reshape(1, DIM),
        SW3, SB3.reshape(1, SINTER))
    posb = jnp.concatenate([pos0, pos1], axis=1).T.reshape(2 * NW, CHUNK)
    xs = _dispatch(x, posb)
    op = _grouped(be.reshape(MAXB), xs, W1, B1, W2, B2, W3, B3)
    posd = posb.reshape(2 * NW * 2, HALF)
    wd = jnp.concatenate([w0, w1], axis=0).reshape(2 * NW * 2, HALF, 16)
    return _combine(ysh, op, posd, wd)


# speedup vs baseline: 1.1786x; 1.1786x over previous
import jax
import jax.numpy as jnp
from jax import lax
from jax.experimental import pallas as pl
from jax.experimental.pallas import tpu as pltpu
from jax.experimental.pallas import tpu_sc as plsc

DIM = 768
INTER = 768
E = 8
SINTER = 1536
NTOK = 2048

BM = 256
MAXB = 23
S = MAXB * BM

NW = 32
CHUNK = NTOK // NW
TB = 16
NB = CHUNK // TB


def _mm_nt(a, b):
    return jax.lax.dot_general(
        a, b, (((1,), (1,)), ((), ())), preferred_element_type=jnp.float32)


def _gate_kernel(x_ref, gw_ref,
                 pos0_ref, pos1_ref, w0_ref, w1_ref, be_ref):
    x = x_ref[...]
    scores = _mm_nt(x, gw_ref[...])
    m = jnp.max(scores, axis=-1, keepdims=True)
    p = jnp.exp(scores - m)
    s = p / jnp.sum(p, axis=-1, keepdims=True)
    lane = jax.lax.broadcasted_iota(jnp.int32, (NTOK, E), 1)
    m1 = jnp.max(s, axis=-1, keepdims=True)
    idx1 = jnp.min(jnp.where(s == m1, lane, E), axis=-1, keepdims=True)
    oh1 = lane == idx1
    s_m = jnp.where(oh1, -jnp.inf, s)
    m2 = jnp.max(s_m, axis=-1, keepdims=True)
    idx2 = jnp.min(jnp.where(s_m == m2, lane, E), axis=-1, keepdims=True)
    oh2 = lane == idx2
    sel = jnp.logical_or(oh1, oh2)

    msk = sel.astype(jnp.float32)
    acc = msk
    shift = 1
    while shift < NTOK:
        acc = acc + jnp.concatenate(
            [jnp.zeros((shift, E), jnp.float32), acc[:NTOK - shift]], axis=0)
        shift *= 2
    rank = (acc - msk).astype(jnp.int32)
    counts = acc[NTOK - 1:NTOK, :].astype(jnp.int32)
    pc = ((counts + (BM - 1)) // BM) * BM
    off = pc
    off = off + jnp.concatenate(
        [jnp.zeros((1, 1), jnp.int32), off[:, :E - 1]], axis=1)
    off = off + jnp.concatenate(
        [jnp.zeros((1, 2), jnp.int32), off[:, :E - 2]], axis=1)
    off = off + jnp.concatenate(
        [jnp.zeros((1, 4), jnp.int32), off[:, :E - 4]], axis=1)
    off = off - pc
    pos = off + rank
    pos0_ref[...] = jnp.sum(jnp.where(oh1, pos, 0), axis=-1, keepdims=True)
    pos1_ref[...] = jnp.sum(jnp.where(oh2, pos, 0), axis=-1, keepdims=True)
    w0 = jnp.sum(jnp.where(oh1, s, 0.0), axis=-1, keepdims=True)
    w1 = jnp.sum(jnp.where(oh2, s, 0.0), axis=-1, keepdims=True)
    w0_ref[...] = jnp.broadcast_to(w0, (NTOK, 16))
    w1_ref[...] = jnp.broadcast_to(w1, (NTOK, 16))
    bstart = jax.lax.broadcasted_iota(jnp.int32, (MAXB, E), 0) * BM
    ends = off + pc
    be = jnp.sum((bstart >= ends).astype(jnp.int32), axis=-1, keepdims=True)
    be_ref[...] = jnp.minimum(be, E - 1)


def _gate(x, gate_w):
    return pl.pallas_call(
        _gate_kernel,
        in_specs=[pl.BlockSpec(x.shape, lambda: (0, 0)),
                  pl.BlockSpec(gate_w.shape, lambda: (0, 0))],
        out_specs=[
            pl.BlockSpec((NTOK, 1), lambda: (0, 0)),
            pl.BlockSpec((NTOK, 1), lambda: (0, 0)),
            pl.BlockSpec((NTOK, 16), lambda: (0, 0)),
            pl.BlockSpec((NTOK, 16), lambda: (0, 0)),
            pl.BlockSpec((MAXB, 1), lambda: (0, 0)),
        ],
        out_shape=[
            jax.ShapeDtypeStruct((NTOK, 1), jnp.int32),
            jax.ShapeDtypeStruct((NTOK, 1), jnp.int32),
            jax.ShapeDtypeStruct((NTOK, 16), jnp.float32),
            jax.ShapeDtypeStruct((NTOK, 16), jnp.float32),
            jax.ShapeDtypeStruct((MAXB, 1), jnp.int32),
        ],
    )(x, gate_w)


def _shared_kernel(x_ref, SW1_ref, SB1_ref, SW2_ref, SB2_ref,
                   SW3_ref, SB3_ref, ysh_ref):
    x = x_ref[...]
    h = jax.nn.silu(_mm_nt(x, SW1_ref[...]) + SB1_ref[...]) * (
        _mm_nt(x, SW3_ref[...]) + SB3_ref[...])
    ysh_ref[...] = _mm_nt(h, SW2_ref[...]) + SB2_ref[...]


def _shared(x, SW1, SB1, SW2, SB2, SW3, SB3):
    args = (x, SW1, SB1, SW2, SB2, SW3, SB3)
    return pl.pallas_call(
        _shared_kernel,
        in_specs=[pl.BlockSpec(a.shape, lambda: (0, 0)) for a in args],
        out_specs=pl.BlockSpec((NTOK, DIM), lambda: (0, 0)),
        out_shape=jax.ShapeDtypeStruct((NTOK, DIM), jnp.float32),
    )(*args)


def _dispatch_kernel(x_hbm, pos0_hbm, pos1_hbm, xs_hbm,
                     xb, idx0, idx1, sem0, sem1):
    wid = lax.axis_index("s") * 2 + lax.axis_index("c")
    base = wid * CHUNK
    pltpu.sync_copy(pos0_hbm.at[pl.ds(base, CHUNK)], idx0)
    pltpu.sync_copy(pos1_hbm.at[pl.ds(base, CHUNK)], idx1)
    pltpu.sync_copy(x_hbm.at[pl.ds(base, CHUNK)], xb)
    c0 = pltpu.make_async_copy(xb, xs_hbm.at[idx0], sem0)
    c1 = pltpu.make_async_copy(xb, xs_hbm.at[idx1], sem1)
    c0.start(); c1.start()
    c0.wait(); c1.wait()


def _dispatch(x, pos0, pos1):
    mesh = plsc.VectorSubcoreMesh(core_axis_name="c", subcore_axis_name="s")
    run = pl.kernel(
        _dispatch_kernel, mesh=mesh,
        out_type=jax.ShapeDtypeStruct((S, DIM), jnp.float32),
        scratch_types=[
            pltpu.VMEM((CHUNK, DIM), jnp.float32),
            pltpu.VMEM((CHUNK,), jnp.int32),
            pltpu.VMEM((CHUNK,), jnp.int32),
            pltpu.SemaphoreType.DMA,
            pltpu.SemaphoreType.DMA,
        ],
    )
    return run(x, pos0, pos1)


def _grouped_kernel(be_ref, xs_ref, W1_ref, B1_ref, W2_ref, B2_ref,
                    W3_ref, B3_ref, out_ref):
    xs = xs_ref[...]
    h = jax.nn.silu(_mm_nt(xs, W1_ref[0]) + B1_ref[0]) * (
        _mm_nt(xs, W3_ref[0]) + B3_ref[0])
    out_ref[...] = _mm_nt(h, W2_ref[0]) + B2_ref[0]


def _grouped(be, xs, W1, B1, W2, B2, W3, B3):
    ew = lambda b, be_ref: (be_ref[b], 0, 0)
    grid_spec = pltpu.PrefetchScalarGridSpec(
        num_scalar_prefetch=1,
        grid=(MAXB,),
        in_specs=[
            pl.BlockSpec((BM, DIM), lambda b, be_ref: (b, 0)),
            pl.BlockSpec((1, INTER, DIM), ew),
            pl.BlockSpec((1, 1, INTER), ew),
            pl.BlockSpec((1, DIM, INTER), ew),
            pl.BlockSpec((1, 1, DIM), ew),
            pl.BlockSpec((1, INTER, DIM), ew),
            pl.BlockSpec((1, 1, INTER), ew),
        ],
        out_specs=pl.BlockSpec((BM, DIM), lambda b, be_ref: (b, 0)),
    )
    return pl.pallas_call(
        _grouped_kernel,
        grid_spec=grid_spec,
        out_shape=jax.ShapeDtypeStruct((S, DIM), jnp.float32),
        compiler_params=pltpu.CompilerParams(
            dimension_semantics=("arbitrary",)),
    )(be, xs, W1, B1.reshape(E, 1, INTER), W2, B2.reshape(E, 1, DIM),
      W3, B3.reshape(E, 1, INTER))


def _combine_kernel(ysh_hbm, op_hbm, pos0_hbm, pos1_hbm, w0_hbm, w1_hbm,
                    y_hbm, acc0, acc1, r00, r01, r10, r11, ob0, ob1,
                    idx0, idx1, wv0, wv1, sema0, sema1, semg0, semg1,
                    semo0, semo1):
    wid = lax.axis_index("s") * 2 + lax.axis_index("c")
    accs = (acc0, acc1)
    r0s = (r00, r01)
    r1s = (r10, r11)
    obs = (ob0, ob1)
    semas = (sema0, sema1)
    semgs = (semg0, semg1)
    semos = (semo0, semo1)

    def start(j, p):
        base = wid * CHUNK + j * TB
        pltpu.sync_copy(pos0_hbm.at[pl.ds(base, TB)], idx0.at[p])
        pltpu.sync_copy(pos1_hbm.at[pl.ds(base, TB)], idx1.at[p])
        pltpu.sync_copy(w0_hbm.at[pl.ds(base, TB)], wv0.at[p])
        pltpu.sync_copy(w1_hbm.at[pl.ds(base, TB)], wv1.at[p])
        pltpu.make_async_copy(
            ysh_hbm.at[pl.ds(base, TB)], accs[p], semas[p]).start()
        pltpu.make_async_copy(op_hbm.at[idx0.at[p]], r0s[p], semgs[p]).start()
        pltpu.make_async_copy(op_hbm.at[idx1.at[p]], r1s[p], semgs[p]).start()

    start(0, 0)
    start(1, 1)
    for j in range(NB):
        p = j % 2
        base = wid * CHUNK + j * TB
        pltpu.make_async_copy(
            ysh_hbm.at[pl.ds(base, TB)], accs[p], semas[p]).wait()
        pltpu.make_async_copy(op_hbm.at[idx0.at[p]], r0s[p], semgs[p]).wait()
        pltpu.make_async_copy(op_hbm.at[idx1.at[p]], r1s[p], semgs[p]).wait()
        if j >= 2:
            pltpu.make_async_copy(
                obs[p], y_hbm.at[pl.ds(base - 2 * TB, TB)], semos[p]).wait()
        acc, r0, r1, ob = accs[p], r0s[p], r1s[p], obs[p]

        def row(r, _):
            g0 = wv0[p, r, pl.ds(0, 16)]
            g1 = wv1[p, r, pl.ds(0, 16)]
            for c in range(DIM // 16):
                sl = pl.ds(c * 16, 16)
                ob[r, sl] = acc[r, sl] + g0 * r0[r, sl] + g1 * r1[r, sl]
            return 0
        lax.fori_loop(0, TB, row, 0)
        pltpu.make_async_copy(
            ob, y_hbm.at[pl.ds(base, TB)], semos[p]).start()
        if j + 2 < NB:
            start(j + 2, p)
    for j in (NB - 2, NB - 1):
        p = j % 2
        base = wid * CHUNK + j * TB
        pltpu.make_async_copy(
            obs[p], y_hbm.at[pl.ds(base, TB)], semos[p]).wait()


def _combine(ysh, op, pos0, pos1, w0, w1):
    mesh = plsc.VectorSubcoreMesh(core_axis_name="c", subcore_axis_name="s")
    run = pl.kernel(
        _combine_kernel, mesh=mesh,
        out_type=jax.ShapeDtypeStruct((NTOK, DIM), jnp.float32),
        scratch_types=[
            pltpu.VMEM((TB, DIM), jnp.float32),
            pltpu.VMEM((TB, DIM), jnp.float32),
            pltpu.VMEM((TB, DIM), jnp.float32),
            pltpu.VMEM((TB, DIM), jnp.float32),
            pltpu.VMEM((TB, DIM), jnp.float32),
            pltpu.VMEM((TB, DIM), jnp.float32),
            pltpu.VMEM((TB, DIM), jnp.float32),
            pltpu.VMEM((TB, DIM), jnp.float32),
            pltpu.VMEM((2, TB), jnp.int32),
            pltpu.VMEM((2, TB), jnp.int32),
            pltpu.VMEM((2, TB, 16), jnp.float32),
            pltpu.VMEM((2, TB, 16), jnp.float32),
            pltpu.SemaphoreType.DMA,
            pltpu.SemaphoreType.DMA,
            pltpu.SemaphoreType.DMA,
            pltpu.SemaphoreType.DMA,
            pltpu.SemaphoreType.DMA,
            pltpu.SemaphoreType.DMA,
        ],
    )
    return run(ysh, op, pos0, pos1, w0, w1)


def kernel(x, gate_w, W1, B1, W2, B2, W3, B3, SW1, SB1, SW2, SB2, SW3, SB3):
    pos0, pos1, w0, w1, be = _gate(x, gate_w)
    pos0 = pos0.reshape(NTOK)
    pos1 = pos1.reshape(NTOK)
    xs = _dispatch(x, pos0, pos1)
    ysh = _shared(x, SW1, SB1.reshape(1, SINTER), SW2, SB2.reshape(1, DIM),
                  SW3, SB3.reshape(1, SINTER))
    op = _grouped(be.reshape(MAXB), xs, W1, B1, W2, B2, W3, B3)
    return _combine(ysh, op, pos0, pos1, w0, w1)


# speedup vs baseline: 1.1995x; 1.0177x over previous
import jax
import jax.numpy as jnp
from jax import lax
from jax.experimental import pallas as pl
from jax.experimental.pallas import tpu as pltpu
from jax.experimental.pallas import tpu_sc as plsc

DIM = 768
INTER = 768
E = 8
SINTER = 1536
NTOK = 2048

BM = 256
MAXB = 23
S = MAXB * BM

NW = 32
CHUNK = NTOK // NW
TB = 16
NB = CHUNK // TB


def _mm_nt(a, b):
    return jax.lax.dot_general(
        a, b, (((1,), (1,)), ((), ())), preferred_element_type=jnp.float32)


def _gate_kernel(x_ref, gw_ref,
                 pos0_ref, pos1_ref, w0_ref, w1_ref, be_ref, bmap_ref):
    x = x_ref[...]
    scores = _mm_nt(x, gw_ref[...])
    m = jnp.max(scores, axis=-1, keepdims=True)
    p = jnp.exp(scores - m)
    s = p / jnp.sum(p, axis=-1, keepdims=True)
    lane = jax.lax.broadcasted_iota(jnp.int32, (NTOK, E), 1)
    m1 = jnp.max(s, axis=-1, keepdims=True)
    idx1 = jnp.min(jnp.where(s == m1, lane, E), axis=-1, keepdims=True)
    oh1 = lane == idx1
    s_m = jnp.where(oh1, -jnp.inf, s)
    m2 = jnp.max(s_m, axis=-1, keepdims=True)
    idx2 = jnp.min(jnp.where(s_m == m2, lane, E), axis=-1, keepdims=True)
    oh2 = lane == idx2
    sel = jnp.logical_or(oh1, oh2)

    msk = sel.astype(jnp.float32)
    acc = msk
    shift = 1
    while shift < NTOK:
        acc = acc + jnp.concatenate(
            [jnp.zeros((shift, E), jnp.float32), acc[:NTOK - shift]], axis=0)
        shift *= 2
    rank = (acc - msk).astype(jnp.int32)
    counts = acc[NTOK - 1:NTOK, :].astype(jnp.int32)
    pc = ((counts + (BM - 1)) // BM) * BM
    off = pc
    off = off + jnp.concatenate(
        [jnp.zeros((1, 1), jnp.int32), off[:, :E - 1]], axis=1)
    off = off + jnp.concatenate(
        [jnp.zeros((1, 2), jnp.int32), off[:, :E - 2]], axis=1)
    off = off + jnp.concatenate(
        [jnp.zeros((1, 4), jnp.int32), off[:, :E - 4]], axis=1)
    off = off - pc
    pos = off + rank
    pos0_ref[...] = jnp.sum(jnp.where(oh1, pos, 0), axis=-1, keepdims=True)
    pos1_ref[...] = jnp.sum(jnp.where(oh2, pos, 0), axis=-1, keepdims=True)
    w0 = jnp.sum(jnp.where(oh1, s, 0.0), axis=-1, keepdims=True)
    w1 = jnp.sum(jnp.where(oh2, s, 0.0), axis=-1, keepdims=True)
    w0_ref[...] = jnp.broadcast_to(w0, (NTOK, 16))
    w1_ref[...] = jnp.broadcast_to(w1, (NTOK, 16))
    bstart = jax.lax.broadcasted_iota(jnp.int32, (MAXB, E), 0) * BM
    ends = off + pc
    be = jnp.sum((bstart >= ends).astype(jnp.int32), axis=-1, keepdims=True)
    be_ref[...] = jnp.minimum(be, E - 1)
    nact = jnp.sum(pc, keepdims=True) // BM
    bids = jax.lax.broadcasted_iota(jnp.int32, (MAXB, 1), 0)
    bmap_ref[...] = jnp.minimum(bids, nact - 1)


def _gate(x, gate_w):
    return pl.pallas_call(
        _gate_kernel,
        in_specs=[pl.BlockSpec(x.shape, lambda: (0, 0)),
                  pl.BlockSpec(gate_w.shape, lambda: (0, 0))],
        out_specs=[
            pl.BlockSpec((NTOK, 1), lambda: (0, 0)),
            pl.BlockSpec((NTOK, 1), lambda: (0, 0)),
            pl.BlockSpec((NTOK, 16), lambda: (0, 0)),
            pl.BlockSpec((NTOK, 16), lambda: (0, 0)),
            pl.BlockSpec((MAXB, 1), lambda: (0, 0)),
            pl.BlockSpec((MAXB, 1), lambda: (0, 0)),
        ],
        out_shape=[
            jax.ShapeDtypeStruct((NTOK, 1), jnp.int32),
            jax.ShapeDtypeStruct((NTOK, 1), jnp.int32),
            jax.ShapeDtypeStruct((NTOK, 16), jnp.float32),
            jax.ShapeDtypeStruct((NTOK, 16), jnp.float32),
            jax.ShapeDtypeStruct((MAXB, 1), jnp.int32),
            jax.ShapeDtypeStruct((MAXB, 1), jnp.int32),
        ],
    )(x, gate_w)


def _shared_kernel(x_ref, SW1_ref, SB1_ref, SW2_ref, SB2_ref,
                   SW3_ref, SB3_ref, ysh_ref):
    x = x_ref[...]
    h = jax.nn.silu(_mm_nt(x, SW1_ref[...]) + SB1_ref[...]) * (
        _mm_nt(x, SW3_ref[...]) + SB3_ref[...])
    ysh_ref[...] = _mm_nt(h, SW2_ref[...]) + SB2_ref[...]


def _shared(x, SW1, SB1, SW2, SB2, SW3, SB3):
    args = (SW1, SB1, SW2, SB2, SW3, SB3)
    tb = NTOK // 4
    return pl.pallas_call(
        _shared_kernel,
        grid=(4,),
        in_specs=[pl.BlockSpec((tb, DIM), lambda i: (i, 0))] +
                 [pl.BlockSpec(a.shape, lambda i: (0, 0)) for a in args],
        out_specs=pl.BlockSpec((tb, DIM), lambda i: (i, 0)),
        out_shape=jax.ShapeDtypeStruct((NTOK, DIM), jnp.float32),
        compiler_params=pltpu.CompilerParams(
            dimension_semantics=("arbitrary",)),
    )(x, *args)


def _dispatch_kernel(x_hbm, pos0_hbm, pos1_hbm, xs_hbm,
                     xb, idx0, idx1, sem0, sem1):
    wid = lax.axis_index("s") * 2 + lax.axis_index("c")
    base = wid * CHUNK
    pltpu.sync_copy(pos0_hbm.at[pl.ds(base, CHUNK)], idx0)
    pltpu.sync_copy(pos1_hbm.at[pl.ds(base, CHUNK)], idx1)
    pltpu.sync_copy(x_hbm.at[pl.ds(base, CHUNK)], xb)
    c0 = pltpu.make_async_copy(xb, xs_hbm.at[idx0], sem0)
    c1 = pltpu.make_async_copy(xb, xs_hbm.at[idx1], sem1)
    c0.start(); c1.start()
    c0.wait(); c1.wait()


def _dispatch(x, pos0, pos1):
    mesh = plsc.VectorSubcoreMesh(core_axis_name="c", subcore_axis_name="s")
    run = pl.kernel(
        _dispatch_kernel, mesh=mesh,
        out_type=jax.ShapeDtypeStruct((S, DIM), jnp.float32),
        scratch_types=[
            pltpu.VMEM((CHUNK, DIM), jnp.float32),
            pltpu.VMEM((CHUNK,), jnp.int32),
            pltpu.VMEM((CHUNK,), jnp.int32),
            pltpu.SemaphoreType.DMA,
            pltpu.SemaphoreType.DMA,
        ],
    )
    return run(x, pos0, pos1)


def _grouped_kernel(bmap_ref, be_ref, xs_ref, W1_ref, B1_ref, W2_ref,
                    B2_ref, W3_ref, B3_ref, out_ref):
    xs = xs_ref[...]
    h = jax.nn.silu(_mm_nt(xs, W1_ref[0]) + B1_ref[0]) * (
        _mm_nt(xs, W3_ref[0]) + B3_ref[0])
    out_ref[...] = _mm_nt(h, W2_ref[0]) + B2_ref[0]


def _grouped(bmap, be, xs, W1, B1, W2, B2, W3, B3):
    ew = lambda b, bmap_ref, be_ref: (be_ref[b], 0, 0)
    bm = lambda b, bmap_ref, be_ref: (bmap_ref[b], 0)
    grid_spec = pltpu.PrefetchScalarGridSpec(
        num_scalar_prefetch=2,
        grid=(MAXB,),
        in_specs=[
            pl.BlockSpec((BM, DIM), bm),
            pl.BlockSpec((1, INTER, DIM), ew),
            pl.BlockSpec((1, 1, INTER), ew),
            pl.BlockSpec((1, DIM, INTER), ew),
            pl.BlockSpec((1, 1, DIM), ew),
            pl.BlockSpec((1, INTER, DIM), ew),
            pl.BlockSpec((1, 1, INTER), ew),
        ],
        out_specs=pl.BlockSpec((BM, DIM), bm),
    )
    return pl.pallas_call(
        _grouped_kernel,
        grid_spec=grid_spec,
        out_shape=jax.ShapeDtypeStruct((S, DIM), jnp.float32),
        compiler_params=pltpu.CompilerParams(
            dimension_semantics=("arbitrary",)),
    )(bmap, be, xs, W1, B1.reshape(E, 1, INTER), W2, B2.reshape(E, 1, DIM),
      W3, B3.reshape(E, 1, INTER))


def _combine_kernel(ysh_hbm, op_hbm, pos0_hbm, pos1_hbm, w0_hbm, w1_hbm,
                    y_hbm, acc0, acc1, r00, r01, r10, r11, ob0, ob1,
                    idx0, idx1, wv0, wv1, sema0, sema1, semg0, semg1,
                    semo0, semo1):
    wid = lax.axis_index("s") * 2 + lax.axis_index("c")
    accs = (acc0, acc1)
    r0s = (r00, r01)
    r1s = (r10, r11)
    obs = (ob0, ob1)
    semas = (sema0, sema1)
    semgs = (semg0, semg1)
    semos = (semo0, semo1)

    def start(j, p):
        base = wid * CHUNK + j * TB
        pltpu.sync_copy(pos0_hbm.at[pl.ds(base, TB)], idx0.at[p])
        pltpu.sync_copy(pos1_hbm.at[pl.ds(base, TB)], idx1.at[p])
        pltpu.sync_copy(w0_hbm.at[pl.ds(base, TB)], wv0.at[p])
        pltpu.sync_copy(w1_hbm.at[pl.ds(base, TB)], wv1.at[p])
        pltpu.make_async_copy(
            ysh_hbm.at[pl.ds(base, TB)], accs[p], semas[p]).start()
        pltpu.make_async_copy(op_hbm.at[idx0.at[p]], r0s[p], semgs[p]).start()
        pltpu.make_async_copy(op_hbm.at[idx1.at[p]], r1s[p], semgs[p]).start()

    start(0, 0)
    start(1, 1)
    for j in range(NB):
        p = j % 2
        base = wid * CHUNK + j * TB
        pltpu.make_async_copy(
            ysh_hbm.at[pl.ds(base, TB)], accs[p], semas[p]).wait()
        pltpu.make_async_copy(op_hbm.at[idx0.at[p]], r0s[p], semgs[p]).wait()
        pltpu.make_async_copy(op_hbm.at[idx1.at[p]], r1s[p], semgs[p]).wait()
        if j >= 2:
            pltpu.make_async_copy(
                obs[p], y_hbm.at[pl.ds(base - 2 * TB, TB)], semos[p]).wait()
        acc, r0, r1, ob = accs[p], r0s[p], r1s[p], obs[p]

        def row(r, _):
            g0 = wv0[p, r, pl.ds(0, 16)]
            g1 = wv1[p, r, pl.ds(0, 16)]
            for c in range(DIM // 16):
                sl = pl.ds(c * 16, 16)
                ob[r, sl] = acc[r, sl] + g0 * r0[r, sl] + g1 * r1[r, sl]
            return 0
        lax.fori_loop(0, TB, row, 0)
        pltpu.make_async_copy(
            ob, y_hbm.at[pl.ds(base, TB)], semos[p]).start()
        if j + 2 < NB:
            start(j + 2, p)
    for j in (NB - 2, NB - 1):
        p = j % 2
        base = wid * CHUNK + j * TB
        pltpu.make_async_copy(
            obs[p], y_hbm.at[pl.ds(base, TB)], semos[p]).wait()


def _combine(ysh, op, pos0, pos1, w0, w1):
    mesh = plsc.VectorSubcoreMesh(core_axis_name="c", subcore_axis_name="s")
    run = pl.kernel(
        _combine_kernel, mesh=mesh,
        out_type=jax.ShapeDtypeStruct((NTOK, DIM), jnp.float32),
        scratch_types=[
            pltpu.VMEM((TB, DIM), jnp.float32),
            pltpu.VMEM((TB, DIM), jnp.float32),
            pltpu.VMEM((TB, DIM), jnp.float32),
            pltpu.VMEM((TB, DIM), jnp.float32),
            pltpu.VMEM((TB, DIM), jnp.float32),
            pltpu.VMEM((TB, DIM), jnp.float32),
            pltpu.VMEM((TB, DIM), jnp.float32),
            pltpu.VMEM((TB, DIM), jnp.float32),
            pltpu.VMEM((2, TB), jnp.int32),
            pltpu.VMEM((2, TB), jnp.int32),
            pltpu.VMEM((2, TB, 16), jnp.float32),
            pltpu.VMEM((2, TB, 16), jnp.float32),
            pltpu.SemaphoreType.DMA,
            pltpu.SemaphoreType.DMA,
            pltpu.SemaphoreType.DMA,
            pltpu.SemaphoreType.DMA,
            pltpu.SemaphoreType.DMA,
            pltpu.SemaphoreType.DMA,
        ],
    )
    return run(ysh, op, pos0, pos1, w0, w1)


def kernel(x, gate_w, W1, B1, W2, B2, W3, B3, SW1, SB1, SW2, SB2, SW3, SB3):
    pos0, pos1, w0, w1, be, bmap = _gate(x, gate_w)
    pos0 = pos0.reshape(NTOK)
    pos1 = pos1.reshape(NTOK)
    xs = _dispatch(x, pos0, pos1)
    ysh = _shared(x, SW1, SB1.reshape(1, SINTER), SW2, SB2.reshape(1, DIM),
                  SW3, SB3.reshape(1, SINTER))
    op = _grouped(bmap.reshape(MAXB), be.reshape(MAXB), xs,
                  W1, B1, W2, B2, W3, B3)
    return _combine(ysh, op, pos0, pos1, w0, w1)


# speedup vs baseline: 1.2182x; 1.0156x over previous
import jax
import jax.numpy as jnp
from jax import lax
from jax.experimental import pallas as pl
from jax.experimental.pallas import tpu as pltpu
from jax.experimental.pallas import tpu_sc as plsc

DIM = 768
INTER = 768
E = 8
SINTER = 1536
NTOK = 2048

BM = 256
MAXB = 23
S = MAXB * BM

NW = 32
CHUNK = NTOK // NW
TB = 16
NB = CHUNK // TB


def _mm_nt(a, b):
    return jax.lax.dot_general(
        a, b, (((1,), (1,)), ((), ())), preferred_element_type=jnp.float32)


def _gate_kernel(x_ref, gw_ref,
                 pos0_ref, pos1_ref, w0_ref, w1_ref, be_ref, bmap_ref):
    x = x_ref[...]
    scores = _mm_nt(x, gw_ref[...])
    m = jnp.max(scores, axis=-1, keepdims=True)
    p = jnp.exp(scores - m)
    s = p / jnp.sum(p, axis=-1, keepdims=True)
    lane = jax.lax.broadcasted_iota(jnp.int32, (NTOK, E), 1)
    m1 = jnp.max(s, axis=-1, keepdims=True)
    idx1 = jnp.min(jnp.where(s == m1, lane, E), axis=-1, keepdims=True)
    oh1 = lane == idx1
    s_m = jnp.where(oh1, -jnp.inf, s)
    m2 = jnp.max(s_m, axis=-1, keepdims=True)
    idx2 = jnp.min(jnp.where(s_m == m2, lane, E), axis=-1, keepdims=True)
    oh2 = lane == idx2
    sel = jnp.logical_or(oh1, oh2)

    msk = sel.astype(jnp.float32)
    acc = msk
    shift = 1
    while shift < NTOK:
        acc = acc + jnp.concatenate(
            [jnp.zeros((shift, E), jnp.float32), acc[:NTOK - shift]], axis=0)
        shift *= 2
    rank = (acc - msk).astype(jnp.int32)
    counts = acc[NTOK - 1:NTOK, :].astype(jnp.int32)
    pc = ((counts + (BM - 1)) // BM) * BM
    off = pc
    off = off + jnp.concatenate(
        [jnp.zeros((1, 1), jnp.int32), off[:, :E - 1]], axis=1)
    off = off + jnp.concatenate(
        [jnp.zeros((1, 2), jnp.int32), off[:, :E - 2]], axis=1)
    off = off + jnp.concatenate(
        [jnp.zeros((1, 4), jnp.int32), off[:, :E - 4]], axis=1)
    off = off - pc
    pos = off + rank
    pos0_ref[...] = jnp.sum(jnp.where(oh1, pos, 0), axis=-1, keepdims=True)
    pos1_ref[...] = jnp.sum(jnp.where(oh2, pos, 0), axis=-1, keepdims=True)
    w0 = jnp.sum(jnp.where(oh1, s, 0.0), axis=-1, keepdims=True)
    w1 = jnp.sum(jnp.where(oh2, s, 0.0), axis=-1, keepdims=True)
    w0_ref[...] = jnp.broadcast_to(w0, (NTOK, 16))
    w1_ref[...] = jnp.broadcast_to(w1, (NTOK, 16))
    bstart = jax.lax.broadcasted_iota(jnp.int32, (MAXB, E), 0) * BM
    ends = off + pc
    be = jnp.sum((bstart >= ends).astype(jnp.int32), axis=-1, keepdims=True)
    be_ref[...] = jnp.minimum(be, E - 1)
    nact = jnp.sum(pc, keepdims=True) // BM
    bids = jax.lax.broadcasted_iota(jnp.int32, (MAXB, 1), 0)
    bmap_ref[...] = jnp.minimum(bids, nact - 1)


def _gate(x, gate_w):
    return pl.pallas_call(
        _gate_kernel,
        in_specs=[pl.BlockSpec(x.shape, lambda: (0, 0)),
                  pl.BlockSpec(gate_w.shape, lambda: (0, 0))],
        out_specs=[
            pl.BlockSpec((NTOK, 1), lambda: (0, 0)),
            pl.BlockSpec((NTOK, 1), lambda: (0, 0)),
            pl.BlockSpec((NTOK, 16), lambda: (0, 0)),
            pl.BlockSpec((NTOK, 16), lambda: (0, 0)),
            pl.BlockSpec((MAXB, 1), lambda: (0, 0)),
            pl.BlockSpec((MAXB, 1), lambda: (0, 0)),
        ],
        out_shape=[
            jax.ShapeDtypeStruct((NTOK, 1), jnp.int32),
            jax.ShapeDtypeStruct((NTOK, 1), jnp.int32),
            jax.ShapeDtypeStruct((NTOK, 16), jnp.float32),
            jax.ShapeDtypeStruct((NTOK, 16), jnp.float32),
            jax.ShapeDtypeStruct((MAXB, 1), jnp.int32),
            jax.ShapeDtypeStruct((MAXB, 1), jnp.int32),
        ],
    )(x, gate_w)


def _shared_kernel(x_ref, yp_ref, SW1_ref, SB1_ref, SW2_ref, SB2_ref,
                   SW3_ref, SB3_ref, y_ref):
    x = x_ref[...]
    h = jax.nn.silu(_mm_nt(x, SW1_ref[...]) + SB1_ref[...]) * (
        _mm_nt(x, SW3_ref[...]) + SB3_ref[...])
    y_ref[...] = _mm_nt(h, SW2_ref[...]) + SB2_ref[...] + yp_ref[...]


def _shared_add(x, yp, SW1, SB1, SW2, SB2, SW3, SB3):
    args = (SW1, SB1, SW2, SB2, SW3, SB3)
    tb = NTOK // 4
    return pl.pallas_call(
        _shared_kernel,
        grid=(4,),
        in_specs=[pl.BlockSpec((tb, DIM), lambda i: (i, 0)),
                  pl.BlockSpec((tb, DIM), lambda i: (i, 0))] +
                 [pl.BlockSpec(a.shape, lambda i: (0, 0)) for a in args],
        out_specs=pl.BlockSpec((tb, DIM), lambda i: (i, 0)),
        out_shape=jax.ShapeDtypeStruct((NTOK, DIM), jnp.float32),
        compiler_params=pltpu.CompilerParams(
            dimension_semantics=("arbitrary",)),
    )(x, yp, *args)


def _dispatch_kernel(x_hbm, pos0_hbm, pos1_hbm, xs_hbm,
                     xb, idx0, idx1, sem0, sem1):
    wid = lax.axis_index("s") * 2 + lax.axis_index("c")
    base = wid * CHUNK
    pltpu.sync_copy(pos0_hbm.at[pl.ds(base, CHUNK)], idx0)
    pltpu.sync_copy(pos1_hbm.at[pl.ds(base, CHUNK)], idx1)
    pltpu.sync_copy(x_hbm.at[pl.ds(base, CHUNK)], xb)
    c0 = pltpu.make_async_copy(xb, xs_hbm.at[idx0], sem0)
    c1 = pltpu.make_async_copy(xb, xs_hbm.at[idx1], sem1)
    c0.start(); c1.start()
    c0.wait(); c1.wait()


def _dispatch(x, pos0, pos1):
    mesh = plsc.VectorSubcoreMesh(core_axis_name="c", subcore_axis_name="s")
    run = pl.kernel(
        _dispatch_kernel, mesh=mesh,
        out_type=jax.ShapeDtypeStruct((S, DIM), jnp.float32),
        scratch_types=[
            pltpu.VMEM((CHUNK, DIM), jnp.float32),
            pltpu.VMEM((CHUNK,), jnp.int32),
            pltpu.VMEM((CHUNK,), jnp.int32),
            pltpu.SemaphoreType.DMA,
            pltpu.SemaphoreType.DMA,
        ],
    )
    return run(x, pos0, pos1)


def _grouped_kernel(bmap_ref, be_ref, xs_ref, W1_ref, B1_ref, W2_ref,
                    B2_ref, W3_ref, B3_ref, out_ref):
    xs = xs_ref[...]
    h = jax.nn.silu(_mm_nt(xs, W1_ref[0]) + B1_ref[0]) * (
        _mm_nt(xs, W3_ref[0]) + B3_ref[0])
    out_ref[...] = _mm_nt(h, W2_ref[0]) + B2_ref[0]


def _grouped(bmap, be, xs, W1, B1, W2, B2, W3, B3):
    ew = lambda b, bmap_ref, be_ref: (be_ref[b], 0, 0)
    bm = lambda b, bmap_ref, be_ref: (bmap_ref[b], 0)
    grid_spec = pltpu.PrefetchScalarGridSpec(
        num_scalar_prefetch=2,
        grid=(MAXB,),
        in_specs=[
            pl.BlockSpec((BM, DIM), bm),
            pl.BlockSpec((1, INTER, DIM), ew),
            pl.BlockSpec((1, 1, INTER), ew),
            pl.BlockSpec((1, DIM, INTER), ew),
            pl.BlockSpec((1, 1, DIM), ew),
            pl.BlockSpec((1, INTER, DIM), ew),
            pl.BlockSpec((1, 1, INTER), ew),
        ],
        out_specs=pl.BlockSpec((BM, DIM), bm),
    )
    return pl.pallas_call(
        _grouped_kernel,
        grid_spec=grid_spec,
        out_shape=jax.ShapeDtypeStruct((S, DIM), jnp.float32),
        compiler_params=pltpu.CompilerParams(
            dimension_semantics=("arbitrary",)),
    )(bmap, be, xs, W1, B1.reshape(E, 1, INTER), W2, B2.reshape(E, 1, DIM),
      W3, B3.reshape(E, 1, INTER))


def _combine_kernel(op_hbm, pos0_hbm, pos1_hbm, w0_hbm, w1_hbm,
                    y_hbm, r00, r01, r10, r11, ob0, ob1,
                    idx0, idx1, wv0, wv1, semg0, semg1,
                    semo0, semo1):
    wid = lax.axis_index("s") * 2 + lax.axis_index("c")
    r0s = (r00, r01)
    r1s = (r10, r11)
    obs = (ob0, ob1)
    semgs = (semg0, semg1)
    semos = (semo0, semo1)

    def start(j, p):
        base = wid * CHUNK + j * TB
        pltpu.sync_copy(pos0_hbm.at[pl.ds(base, TB)], idx0.at[p])
        pltpu.sync_copy(pos1_hbm.at[pl.ds(base, TB)], idx1.at[p])
        pltpu.sync_copy(w0_hbm.at[pl.ds(base, TB)], wv0.at[p])
        pltpu.sync_copy(w1_hbm.at[pl.ds(base, TB)], wv1.at[p])
        pltpu.make_async_copy(op_hbm.at[idx0.at[p]], r0s[p], semgs[p]).start()
        pltpu.make_async_copy(op_hbm.at[idx1.at[p]], r1s[p], semgs[p]).start()

    start(0, 0)
    start(1, 1)
    for j in range(NB):
        p = j % 2
        base = wid * CHUNK + j * TB
        pltpu.make_async_copy(op_hbm.at[idx0.at[p]], r0s[p], semgs[p]).wait()
        pltpu.make_async_copy(op_hbm.at[idx1.at[p]], r1s[p], semgs[p]).wait()
        if j >= 2:
            pltpu.make_async_copy(
                obs[p], y_hbm.at[pl.ds(base - 2 * TB, TB)], semos[p]).wait()
        r0, r1, ob = r0s[p], r1s[p], obs[p]

        def row(r, _):
            g0 = wv0[p, r, pl.ds(0, 16)]
            g1 = wv1[p, r, pl.ds(0, 16)]
            for c in range(DIM // 16):
                sl = pl.ds(c * 16, 16)
                ob[r, sl] = g0 * r0[r, sl] + g1 * r1[r, sl]
            return 0
        lax.fori_loop(0, TB, row, 0)
        pltpu.make_async_copy(
            ob, y_hbm.at[pl.ds(base, TB)], semos[p]).start()
        if j + 2 < NB:
            start(j + 2, p)
    for j in (NB - 2, NB - 1):
        p = j % 2
        base = wid * CHUNK + j * TB
        pltpu.make_async_copy(
            obs[p], y_hbm.at[pl.ds(base, TB)], semos[p]).wait()


def _combine(op, pos0, pos1, w0, w1):
    mesh = plsc.VectorSubcoreMesh(core_axis_name="c", subcore_axis_name="s")
    run = pl.kernel(
        _combine_kernel, mesh=mesh,
        out_type=jax.ShapeDtypeStruct((NTOK, DIM), jnp.float32),
        scratch_types=[
            pltpu.VMEM((TB, DIM), jnp.float32),
            pltpu.VMEM((TB, DIM), jnp.float32),
            pltpu.VMEM((TB, DIM), jnp.float32),
            pltpu.VMEM((TB, DIM), jnp.float32),
            pltpu.VMEM((TB, DIM), jnp.float32),
            pltpu.VMEM((TB, DIM), jnp.float32),
            pltpu.VMEM((2, TB), jnp.int32),
            pltpu.VMEM((2, TB), jnp.int32),
            pltpu.VMEM((2, TB, 16), jnp.float32),
            pltpu.VMEM((2, TB, 16), jnp.float32),
            pltpu.SemaphoreType.DMA,
            pltpu.SemaphoreType.DMA,
            pltpu.SemaphoreType.DMA,
            pltpu.SemaphoreType.DMA,
        ],
    )
    return run(op, pos0, pos1, w0, w1)


def kernel(x, gate_w, W1, B1, W2, B2, W3, B3, SW1, SB1, SW2, SB2, SW3, SB3):
    pos0, pos1, w0, w1, be, bmap = _gate(x, gate_w)
    pos0 = pos0.reshape(NTOK)
    pos1 = pos1.reshape(NTOK)
    xs = _dispatch(x, pos0, pos1)
    op = _grouped(bmap.reshape(MAXB), be.reshape(MAXB), xs,
                  W1, B1, W2, B2, W3, B3)
    yp = _combine(op, pos0, pos1, w0, w1)
    return _shared_add(x, yp, SW1, SB1.reshape(1, SINTER), SW2,
                       SB2.reshape(1, DIM), SW3, SB3.reshape(1, SINTER))


# speedup vs baseline: 1.2405x; 1.0183x over previous
import jax
import jax.numpy as jnp
from jax import lax
from jax.experimental import pallas as pl
from jax.experimental.pallas import tpu as pltpu
from jax.experimental.pallas import tpu_sc as plsc

DIM = 768
INTER = 768
E = 8
SINTER = 1536
NTOK = 2048

BM = 256
MAXB = 23
S = MAXB * BM

NW = 32
CHUNK = NTOK // NW
TB = 16
NB = CHUNK // TB


def _mm_nt(a, b):
    return jax.lax.dot_general(
        a, b, (((1,), (1,)), ((), ())), preferred_element_type=jnp.float32)


def _gate_kernel(x_ref, gw_ref,
                 pos0_ref, pos1_ref, wp_ref, be_ref, bmap_ref):
    x = x_ref[...]
    scores = _mm_nt(x, gw_ref[...])
    m = jnp.max(scores, axis=-1, keepdims=True)
    p = jnp.exp(scores - m)
    s = p / jnp.sum(p, axis=-1, keepdims=True)
    lane = jax.lax.broadcasted_iota(jnp.int32, (NTOK, E), 1)
    m1 = jnp.max(s, axis=-1, keepdims=True)
    idx1 = jnp.min(jnp.where(s == m1, lane, E), axis=-1, keepdims=True)
    oh1 = lane == idx1
    s_m = jnp.where(oh1, -jnp.inf, s)
    m2 = jnp.max(s_m, axis=-1, keepdims=True)
    idx2 = jnp.min(jnp.where(s_m == m2, lane, E), axis=-1, keepdims=True)
    oh2 = lane == idx2
    sel = jnp.logical_or(oh1, oh2)

    msk = sel.astype(jnp.float32)
    acc = msk
    shift = 1
    while shift < NTOK:
        acc = acc + jnp.concatenate(
            [jnp.zeros((shift, E), jnp.float32), acc[:NTOK - shift]], axis=0)
        shift *= 2
    rank = (acc - msk).astype(jnp.int32)
    counts = acc[NTOK - 1:NTOK, :].astype(jnp.int32)
    pc = ((counts + (BM - 1)) // BM) * BM
    off = pc
    off = off + jnp.concatenate(
        [jnp.zeros((1, 1), jnp.int32), off[:, :E - 1]], axis=1)
    off = off + jnp.concatenate(
        [jnp.zeros((1, 2), jnp.int32), off[:, :E - 2]], axis=1)
    off = off + jnp.concatenate(
        [jnp.zeros((1, 4), jnp.int32), off[:, :E - 4]], axis=1)
    off = off - pc
    pos = off + rank
    pos0_ref[...] = jnp.sum(jnp.where(oh1, pos, 0), axis=-1, keepdims=True)
    pos1_ref[...] = jnp.sum(jnp.where(oh2, pos, 0), axis=-1, keepdims=True)
    w0 = jnp.sum(jnp.where(oh1, s, 0.0), axis=-1, keepdims=True)
    w1 = jnp.sum(jnp.where(oh2, s, 0.0), axis=-1, keepdims=True)
    wp_ref[...] = jnp.concatenate(
        [jnp.broadcast_to(w0, (NTOK, 16)), jnp.broadcast_to(w1, (NTOK, 16)),
         jnp.zeros((NTOK, 96), jnp.float32)], axis=1)
    bstart = jax.lax.broadcasted_iota(jnp.int32, (MAXB, E), 0) * BM
    ends = off + pc
    be = jnp.sum((bstart >= ends).astype(jnp.int32), axis=-1, keepdims=True)
    be_ref[...] = jnp.minimum(be, E - 1)
    nact = jnp.sum(pc, keepdims=True) // BM
    bids = jax.lax.broadcasted_iota(jnp.int32, (MAXB, 1), 0)
    bmap_ref[...] = jnp.minimum(bids, nact - 1)


def _gate(x, gate_w):
    return pl.pallas_call(
        _gate_kernel,
        in_specs=[pl.BlockSpec(x.shape, lambda: (0, 0)),
                  pl.BlockSpec(gate_w.shape, lambda: (0, 0))],
        out_specs=[
            pl.BlockSpec((NTOK, 1), lambda: (0, 0)),
            pl.BlockSpec((NTOK, 1), lambda: (0, 0)),
            pl.BlockSpec((NTOK, 128), lambda: (0, 0)),
            pl.BlockSpec((MAXB, 1), lambda: (0, 0)),
            pl.BlockSpec((MAXB, 1), lambda: (0, 0)),
        ],
        out_shape=[
            jax.ShapeDtypeStruct((NTOK, 1), jnp.int32),
            jax.ShapeDtypeStruct((NTOK, 1), jnp.int32),
            jax.ShapeDtypeStruct((NTOK, 128), jnp.float32),
            jax.ShapeDtypeStruct((MAXB, 1), jnp.int32),
            jax.ShapeDtypeStruct((MAXB, 1), jnp.int32),
        ],
    )(x, gate_w)


def _shared_kernel(x_ref, yp_ref, SW1_ref, SB1_ref, SW2_ref, SB2_ref,
                   SW3_ref, SB3_ref, y_ref):
    x = x_ref[...]
    h = jax.nn.silu(_mm_nt(x, SW1_ref[...]) + SB1_ref[...]) * (
        _mm_nt(x, SW3_ref[...]) + SB3_ref[...])
    y_ref[...] = _mm_nt(h, SW2_ref[...]) + SB2_ref[...] + yp_ref[...]


def _shared_add(x, yp, SW1, SB1, SW2, SB2, SW3, SB3):
    args = (SW1, SB1, SW2, SB2, SW3, SB3)
    tb = NTOK // 4
    return pl.pallas_call(
        _shared_kernel,
        grid=(4,),
        in_specs=[pl.BlockSpec((tb, DIM), lambda i: (i, 0)),
                  pl.BlockSpec((tb, DIM), lambda i: (i, 0))] +
                 [pl.BlockSpec(a.shape, lambda i: (0, 0)) for a in args],
        out_specs=pl.BlockSpec((tb, DIM), lambda i: (i, 0)),
        out_shape=jax.ShapeDtypeStruct((NTOK, DIM), jnp.float32),
        compiler_params=pltpu.CompilerParams(
            dimension_semantics=("arbitrary",)),
    )(x, yp, *args)


def _dispatch_kernel(x_hbm, pos0_hbm, pos1_hbm, xs_hbm,
                     xb, idx0, idx1, sem0, sem1):
    wid = lax.axis_index("s") * 2 + lax.axis_index("c")
    base = wid * CHUNK
    pltpu.sync_copy(pos0_hbm.at[pl.ds(base, CHUNK)], idx0)
    pltpu.sync_copy(pos1_hbm.at[pl.ds(base, CHUNK)], idx1)
    pltpu.sync_copy(x_hbm.at[pl.ds(base, CHUNK)], xb)
    c0 = pltpu.make_async_copy(xb, xs_hbm.at[idx0], sem0)
    c1 = pltpu.make_async_copy(xb, xs_hbm.at[idx1], sem1)
    c0.start(); c1.start()
    c0.wait(); c1.wait()


def _dispatch(x, pos0, pos1):
    mesh = plsc.VectorSubcoreMesh(core_axis_name="c", subcore_axis_name="s")
    run = pl.kernel(
        _dispatch_kernel, mesh=mesh,
        out_type=jax.ShapeDtypeStruct((S, DIM), jnp.float32),
        scratch_types=[
            pltpu.VMEM((CHUNK, DIM), jnp.float32),
            pltpu.VMEM((CHUNK,), jnp.int32),
            pltpu.VMEM((CHUNK,), jnp.int32),
            pltpu.SemaphoreType.DMA,
            pltpu.SemaphoreType.DMA,
        ],
    )
    return run(x, pos0, pos1)


def _grouped_kernel(bmap_ref, be_ref, xs_ref, W1_ref, B1_ref, W2_ref,
                    B2_ref, W3_ref, B3_ref, out_ref):
    xs = xs_ref[...]
    h = jax.nn.silu(_mm_nt(xs, W1_ref[0]) + B1_ref[0]) * (
        _mm_nt(xs, W3_ref[0]) + B3_ref[0])
    out_ref[...] = _mm_nt(h, W2_ref[0]) + B2_ref[0]


def _grouped(bmap, be, xs, W1, B1, W2, B2, W3, B3):
    ew = lambda b, bmap_ref, be_ref: (be_ref[b], 0, 0)
    bm = lambda b, bmap_ref, be_ref: (bmap_ref[b], 0)
    grid_spec = pltpu.PrefetchScalarGridSpec(
        num_scalar_prefetch=2,
        grid=(MAXB,),
        in_specs=[
            pl.BlockSpec((BM, DIM), bm),
            pl.BlockSpec((1, INTER, DIM), ew),
            pl.BlockSpec((1, 1, INTER), ew),
            pl.BlockSpec((1, DIM, INTER), ew),
            pl.BlockSpec((1, 1, DIM), ew),
            pl.BlockSpec((1, INTER, DIM), ew),
            pl.BlockSpec((1, 1, INTER), ew),
        ],
        out_specs=pl.BlockSpec((BM, DIM), bm),
    )
    return pl.pallas_call(
        _grouped_kernel,
        grid_spec=grid_spec,
        out_shape=jax.ShapeDtypeStruct((S, DIM), jnp.float32),
        compiler_params=pltpu.CompilerParams(
            dimension_semantics=("arbitrary",)),
    )(bmap, be, xs, W1, B1.reshape(E, 1, INTER), W2, B2.reshape(E, 1, DIM),
      W3, B3.reshape(E, 1, INTER))


def _combine_kernel(op_hbm, pos0_hbm, pos1_hbm, wp_hbm,
                    y_hbm, r00, r01, r10, r11, ob0, ob1,
                    idx0, idx1, wv, semg0, semg1,
                    semo0, semo1):
    wid = lax.axis_index("s") * 2 + lax.axis_index("c")
    r0s = (r00, r01)
    r1s = (r10, r11)
    obs = (ob0, ob1)
    semgs = (semg0, semg1)
    semos = (semo0, semo1)

    def start(j, p):
        base = wid * CHUNK + j * TB
        pltpu.sync_copy(pos0_hbm.at[pl.ds(base, TB)], idx0.at[p])
        pltpu.sync_copy(pos1_hbm.at[pl.ds(base, TB)], idx1.at[p])
        pltpu.sync_copy(wp_hbm.at[pl.ds(base, TB)], wv.at[p])
        pltpu.make_async_copy(op_hbm.at[idx0.at[p]], r0s[p], semgs[p]).start()
        pltpu.make_async_copy(op_hbm.at[idx1.at[p]], r1s[p], semgs[p]).start()

    start(0, 0)
    start(1, 1)
    for j in range(NB):
        p = j % 2
        base = wid * CHUNK + j * TB
        pltpu.make_async_copy(op_hbm.at[idx0.at[p]], r0s[p], semgs[p]).wait()
        pltpu.make_async_copy(op_hbm.at[idx1.at[p]], r1s[p], semgs[p]).wait()
        if j >= 2:
            pltpu.make_async_copy(
                obs[p], y_hbm.at[pl.ds(base - 2 * TB, TB)], semos[p]).wait()
        r0, r1, ob = r0s[p], r1s[p], obs[p]

        def row(r, _):
            g0 = wv[p, r, pl.ds(0, 16)]
            g1 = wv[p, r, pl.ds(16, 16)]
            for c in range(DIM // 16):
                sl = pl.ds(c * 16, 16)
                ob[r, sl] = g0 * r0[r, sl] + g1 * r1[r, sl]
            return 0
        lax.fori_loop(0, TB, row, 0)
        pltpu.make_async_copy(
            ob, y_hbm.at[pl.ds(base, TB)], semos[p]).start()
        if j + 2 < NB:
            start(j + 2, p)
    for j in (NB - 2, NB - 1):
        p = j % 2
        base = wid * CHUNK + j * TB
        pltpu.make_async_copy(
            obs[p], y_hbm.at[pl.ds(base, TB)], semos[p]).wait()


def _combine(op, pos0, pos1, wp):
    mesh = plsc.VectorSubcoreMesh(core_axis_name="c", subcore_axis_name="s")
    run = pl.kernel(
        _combine_kernel, mesh=mesh,
        out_type=jax.ShapeDtypeStruct((NTOK, DIM), jnp.float32),
        scratch_types=[
            pltpu.VMEM((TB, DIM), jnp.float32),
            pltpu.VMEM((TB, DIM), jnp.float32),
            pltpu.VMEM((TB, DIM), jnp.float32),
            pltpu.VMEM((TB, DIM), jnp.float32),
            pltpu.VMEM((TB, DIM), jnp.float32),
            pltpu.VMEM((TB, DIM), jnp.float32),
            pltpu.VMEM((2, TB), jnp.int32),
            pltpu.VMEM((2, TB), jnp.int32),
            pltpu.VMEM((2, TB, 128), jnp.float32),
            pltpu.SemaphoreType.DMA,
            pltpu.SemaphoreType.DMA,
            pltpu.SemaphoreType.DMA,
            pltpu.SemaphoreType.DMA,
        ],
    )
    return run(op, pos0, pos1, wp)


def kernel(x, gate_w, W1, B1, W2, B2, W3, B3, SW1, SB1, SW2, SB2, SW3, SB3):
    pos0, pos1, wp, be, bmap = _gate(x, gate_w)
    pos0 = pos0.reshape(NTOK)
    pos1 = pos1.reshape(NTOK)
    xs = _dispatch(x, pos0, pos1)
    op = _grouped(bmap.reshape(MAXB), be.reshape(MAXB), xs,
                  W1, B1, W2, B2, W3, B3)
    yp = _combine(op, pos0, pos1, wp)
    return _shared_add(x, yp, SW1, SB1.reshape(1, SINTER), SW2,
                       SB2.reshape(1, DIM), SW3, SB3.reshape(1, SINTER))
